# baseline (speedup 1.0000x reference)
import jax
import jax.numpy as jnp
from jax.experimental import pallas as pl
from jax.experimental.pallas import tpu as pltpu

_N_CHUNKS = 4


def _copy_body(src, dst, sems):
    s3 = src.reshape(_N_CHUNKS, 8, src.shape[1])
    d3 = dst.reshape(_N_CHUNKS, 8, src.shape[1])
    copies = [
        pltpu.make_async_copy(s3.at[i], d3.at[i], sems.at[i])
        for i in range(_N_CHUNKS)
    ]
    for cp in copies:
        cp.start()
    for cp in copies:
        cp.wait()


def kernel(embeds):
    t = embeds.T
    out = pl.pallas_call(
        _copy_body,
        out_shape=jax.ShapeDtypeStruct(t.shape, t.dtype),
        in_specs=[pl.BlockSpec(memory_space=pltpu.MemorySpace.HBM)],
        out_specs=pl.BlockSpec(memory_space=pltpu.MemorySpace.HBM),
        scratch_shapes=[pltpu.SemaphoreType.DMA((_N_CHUNKS,))],
    )(t)
    return out.T


# speedup vs baseline: 48.5369x; 48.5369x over previous
import jax
import jax.numpy as jnp
from jax.experimental import pallas as pl
from jax.experimental.pallas import tpu as pltpu

_LANES_TOTAL = 1_000_000
_W = 32_768
_N_FULL = _LANES_TOTAL // _W
_TAIL = _LANES_TOTAL - _N_FULL * _W
_K = 32
_DI = 16


def _copy_body(src, dst, bufs, tbufs, in_sems, out_sems, tin_sems, tout_sems):
    s3 = src.reshape(4, 8, _LANES_TOTAL)
    d3 = dst.reshape(4, 8, _LANES_TOTAL)

    chunks = [(i, j * _W) for i in range(4) for j in range(_N_FULL)]
    n_chunks = len(chunks)

    def in_copy(c):
        i, off = chunks[c]
        return pltpu.make_async_copy(
            s3.at[i, :, pl.ds(off, _W)], bufs.at[c % _K], in_sems.at[c % _K]
        )

    def out_copy(c):
        i, off = chunks[c]
        return pltpu.make_async_copy(
            bufs.at[c % _K], d3.at[i, :, pl.ds(off, _W)], out_sems.at[c % _K]
        )

    def tin_copy(i):
        return pltpu.make_async_copy(
            s3.at[i, :, pl.ds(_N_FULL * _W, _TAIL)], tbufs.at[i], tin_sems.at[i]
        )

    def tout_copy(i):
        return pltpu.make_async_copy(
            tbufs.at[i], d3.at[i, :, pl.ds(_N_FULL * _W, _TAIL)], tout_sems.at[i]
        )

    for i in range(4):
        tin_copy(i).start()
    for c in range(min(_DI, n_chunks)):
        in_copy(c).start()
    waited = set()
    for c in range(n_chunks):
        in_copy(c).wait()
        out_copy(c).start()
        p = c + _DI
        if p < n_chunks:
            if p >= _K:
                out_copy(p - _K).wait()
                waited.add(p - _K)
            in_copy(p).start()
    for i in range(4):
        tin_copy(i).wait()
        tout_copy(i).start()
    for c in range(n_chunks):
        if c not in waited:
            out_copy(c).wait()
    for i in range(4):
        tout_copy(i).wait()


def kernel(embeds):
    t = embeds.T
    out = pl.pallas_call(
        _copy_body,
        out_shape=jax.ShapeDtypeStruct(t.shape, t.dtype),
        in_specs=[pl.BlockSpec(memory_space=pltpu.MemorySpace.HBM)],
        out_specs=pl.BlockSpec(memory_space=pltpu.MemorySpace.HBM),
        scratch_shapes=[
            pltpu.VMEM((_K, 8, _W), jnp.float32),
            pltpu.VMEM((4, 8, _TAIL), jnp.float32),
            pltpu.SemaphoreType.DMA((_K,)),
            pltpu.SemaphoreType.DMA((_K,)),
            pltpu.SemaphoreType.DMA((4,)),
            pltpu.SemaphoreType.DMA((4,)),
        ],
    )(t)
    return out.T


# speedup vs baseline: 48.6278x; 1.0019x over previous
import jax
import jax.numpy as jnp
from jax.experimental import pallas as pl
from jax.experimental.pallas import tpu as pltpu

_LANES_TOTAL = 1_000_000
_W = 65_536
_N_FULL = _LANES_TOTAL // _W
_TAIL = _LANES_TOTAL - _N_FULL * _W
_K = 24
_DI = 12


def _copy_body(src, dst, bufs, tbufs, in_sems, out_sems, tin_sems, tout_sems):
    s3 = src.reshape(4, 8, _LANES_TOTAL)
    d3 = dst.reshape(4, 8, _LANES_TOTAL)

    chunks = [(i, j * _W) for i in range(4) for j in range(_N_FULL)]
    n_chunks = len(chunks)

    def in_copy(c):
        i, off = chunks[c]
        return pltpu.make_async_copy(
            s3.at[i, :, pl.ds(off, _W)], bufs.at[c % _K], in_sems.at[c % _K]
        )

    def out_copy(c):
        i, off = chunks[c]
        return pltpu.make_async_copy(
            bufs.at[c % _K], d3.at[i, :, pl.ds(off, _W)], out_sems.at[c % _K]
        )

    def tin_copy(i):
        return pltpu.make_async_copy(
            s3.at[i, :, pl.ds(_N_FULL * _W, _TAIL)], tbufs.at[i], tin_sems.at[i]
        )

    def tout_copy(i):
        return pltpu.make_async_copy(
            tbufs.at[i], d3.at[i, :, pl.ds(_N_FULL * _W, _TAIL)], tout_sems.at[i]
        )

    for i in range(4):
        tin_copy(i).start()
    for c in range(min(_DI, n_chunks)):
        in_copy(c).start()
    waited = set()
    for c in range(n_chunks):
        in_copy(c).wait()
        out_copy(c).start()
        p = c + _DI
        if p < n_chunks:
            if p >= _K:
                out_copy(p - _K).wait()
                waited.add(p - _K)
            in_copy(p).start()
    for i in range(4):
        tin_copy(i).wait()
        tout_copy(i).start()
    for c in range(n_chunks):
        if c not in waited:
            out_copy(c).wait()
    for i in range(4):
        tout_copy(i).wait()


def kernel(embeds):
    t = embeds.T
    out = pl.pallas_call(
        _copy_body,
        out_shape=jax.ShapeDtypeStruct(t.shape, t.dtype),
        in_specs=[pl.BlockSpec(memory_space=pltpu.MemorySpace.HBM)],
        out_specs=pl.BlockSpec(memory_space=pltpu.MemorySpace.HBM),
        scratch_shapes=[
            pltpu.VMEM((_K, 8, _W), jnp.float32),
            pltpu.VMEM((4, 8, _TAIL), jnp.float32),
            pltpu.SemaphoreType.DMA((_K,)),
            pltpu.SemaphoreType.DMA((_K,)),
            pltpu.SemaphoreType.DMA((4,)),
            pltpu.SemaphoreType.DMA((4,)),
        ],
    )(t)
    return out.T
